# baseline (speedup 1.0000x reference)
import functools

import jax
import jax.numpy as jnp
from jax import lax
from jax.experimental import pallas as pl
from jax.experimental.pallas import tpu as pltpu
from jax.experimental.pallas import tpu_sc as plsc

_info = plsc.get_sparse_core_info()
_NC, _NS = _info.num_cores, _info.num_subcores
_NW = _NC * _NS

_CHUNK = 1024


def _make_lookup(total, dim):
    assert total % _NW == 0
    per_w = total // _NW
    assert per_w % _CHUNK == 0
    n_chunks = per_w // _CHUNK

    mesh = plsc.VectorSubcoreMesh(core_axis_name="c", subcore_axis_name="s")

    @functools.partial(
        pl.kernel,
        mesh=mesh,
        out_type=jax.ShapeDtypeStruct((total, dim), jnp.float32),
        scratch_types=[
            pltpu.VMEM((_CHUNK,), jnp.int32),
            pltpu.VMEM((_CHUNK, dim), jnp.float32),
            pltpu.SemaphoreType.DMA,
        ],
        compiler_params=pltpu.CompilerParams(use_tc_tiling_on_sc=False),
    )
    def lookup(table_hbm, idx_hbm, out_hbm, idx_v, rows_v, sem):
        wid = lax.axis_index("s") * _NC + lax.axis_index("c")
        base = wid * per_w

        def step(i, carry):
            off = base + i * _CHUNK
            pltpu.sync_copy(idx_hbm.at[pl.ds(off, _CHUNK)], idx_v)
            pltpu.async_copy(table_hbm.at[idx_v], rows_v, sem).wait()
            pltpu.sync_copy(rows_v, out_hbm.at[pl.ds(off, _CHUNK)])
            return carry

        lax.fori_loop(0, n_chunks, step, 0)

    return lookup


def kernel(x, table):
    batch, seq = x.shape
    total = batch * seq
    dim = table.shape[1]
    x_flat = x.reshape(total).astype(jnp.int32)
    out = _make_lookup(total, dim)(table, x_flat)
    return out.reshape(batch, seq, dim)


# speedup vs baseline: 1.0293x; 1.0293x over previous
import functools

import jax
import jax.numpy as jnp
from jax import lax
from jax.experimental import pallas as pl
from jax.experimental.pallas import tpu as pltpu
from jax.experimental.pallas import tpu_sc as plsc

_info = plsc.get_sparse_core_info()
_NC, _NS = _info.num_cores, _info.num_subcores
_NW = _NC * _NS

_CHUNK = 800
_NBUF = 2


def _make_lookup(total, dim):
    assert total % _NW == 0
    per_w = total // _NW
    assert per_w % _CHUNK == 0
    n_chunks = per_w // _CHUNK
    assert n_chunks > _NBUF

    mesh = plsc.VectorSubcoreMesh(core_axis_name="c", subcore_axis_name="s")

    @functools.partial(
        pl.kernel,
        mesh=mesh,
        out_type=jax.ShapeDtypeStruct((total, dim), jnp.float32),
        scratch_types=[
            pltpu.VMEM((_NBUF, _CHUNK), jnp.int32),
            pltpu.VMEM((_NBUF, _CHUNK, dim), jnp.float32),
            pltpu.SemaphoreType.DMA((_NBUF,)),
            pltpu.SemaphoreType.DMA((_NBUF,)),
            pltpu.SemaphoreType.DMA((_NBUF,)),
        ],
        compiler_params=pltpu.CompilerParams(use_tc_tiling_on_sc=False),
    )
    def lookup(table_hbm, idx_hbm, out_hbm, idx_v, rows_v, s_idx, s_gat, s_out):
        wid = lax.axis_index("s") * _NC + lax.axis_index("c")
        base = wid * per_w

        def idx_load(c, b):
            pltpu.async_copy(
                idx_hbm.at[pl.ds(base + c * _CHUNK, _CHUNK)], idx_v.at[b],
                s_idx.at[b])

        def idx_wait(b):
            pltpu.make_async_copy(
                idx_hbm.at[pl.ds(0, _CHUNK)], idx_v.at[b], s_idx.at[b]).wait()

        def gather(b):
            pltpu.async_copy(table_hbm.at[idx_v.at[b]], rows_v.at[b],
                             s_gat.at[b])

        def gather_wait(b):
            pltpu.make_async_copy(
                table_hbm.at[idx_v.at[b]], rows_v.at[b], s_gat.at[b]).wait()

        def write(c, b):
            pltpu.async_copy(
                rows_v.at[b], out_hbm.at[pl.ds(base + c * _CHUNK, _CHUNK)],
                s_out.at[b])

        def write_wait(b):
            pltpu.make_async_copy(
                rows_v.at[b], out_hbm.at[pl.ds(0, _CHUNK)], s_out.at[b]).wait()

        for b in range(_NBUF):
            idx_load(b, b)

        def step(g, carry):
            for b in range(_NBUF):
                c = g * _NBUF + b
                idx_wait(b)

                @pl.when(g > 0)
                def _():
                    write_wait(b)

                gather(b)
                gather_wait(b)
                write(c, b)

                @pl.when(c + _NBUF < n_chunks)
                def _():
                    idx_load(c + _NBUF, b)

            return carry

        lax.fori_loop(0, n_chunks // _NBUF, step, 0)

        for b in range(_NBUF):
            write_wait(b)

    return lookup


def kernel(x, table):
    batch, seq = x.shape
    total = batch * seq
    dim = table.shape[1]
    x_flat = x.reshape(total).astype(jnp.int32)
    out = _make_lookup(total, dim)(table, x_flat)
    return out.reshape(batch, seq, dim)
